# baseline (speedup 1.0000x reference)
import functools

import jax
import jax.numpy as jnp
from jax import lax
from jax.experimental import pallas as pl
from jax.experimental.pallas import tpu as pltpu
from jax.experimental.pallas import tpu_sc as plsc

NC = 2
NS = 16
NW = NC * NS
LN_EPS = 1e-5

def _sc_mesh():
    return plsc.VectorSubcoreMesh(
        core_axis_name="c", subcore_axis_name="s", num_cores=NC,
        num_subcores=NS,
    )


def _pad_rows(n):
    q = 128 * NS
    return ((n + q - 1) // q) * q


def _chunk_indices(src, dst, n, npad):
    ek = 128
    e = src.shape[0]
    s_per_w = -(-e // (NW * ek))
    if s_per_w % 2:
        s_per_w += 1
    epad = NW * s_per_w * ek
    pad = epad - e
    padr = jnp.arange(pad, dtype=jnp.int32) % (npad - n) + n
    src_p = jnp.concatenate([src, jnp.zeros((pad,), jnp.int32)])
    dst_p = jnp.concatenate([dst, padr])

    def lay(a):
        return a.reshape(s_per_w, NW, ek).transpose(1, 0, 2).reshape(
            NW * s_per_w, ek)

    return lay(src_p), lay(dst_p), s_per_w


def _sc_degree(dst2d, s_per_w, n):
    npad = _pad_rows(n)
    rows = npad // NS
    zrows = 128
    ones = jnp.ones((128, 128), jnp.float32)
    zeros = jnp.zeros((zrows, 128), jnp.float32)
    fire = 8
    assert s_per_w % fire == 0

    @functools.partial(
        pl.kernel,
        out_type=jax.ShapeDtypeStruct((NC, npad, 128), jnp.float32),
        mesh=_sc_mesh(),
        scratch_types=[
            pltpu.VMEM_SHARED((npad, 128), jnp.float32),
            pltpu.VMEM((128, 128), jnp.float32),
            pltpu.VMEM((s_per_w, 128), jnp.int32),
            pltpu.SemaphoreType.DMA,
        ],
    )
    def deg_kernel(dst_hbm, ones_hbm, zeros_hbm, out_hbm, acc_sh, ones_v,
                   idx_v, sem):
        c = lax.axis_index("c")
        s = lax.axis_index("s")
        wid = c * NS + s
        pltpu.sync_copy(ones_hbm, ones_v)
        pltpu.sync_copy(dst_hbm.at[pl.ds(wid * s_per_w, s_per_w)], idx_v)

        @pl.loop(0, rows // zrows)
        def _(j):
            pltpu.sync_copy(zeros_hbm,
                            acc_sh.at[pl.ds(s * rows + j * zrows, zrows)])

        plsc.subcore_barrier()

        @pl.loop(0, s_per_w, step=fire)
        def _(j):
            @pl.loop(0, fire)
            def _(b):
                pltpu.async_copy(ones_v, acc_sh.at[idx_v.at[j + b]], sem,
                                 add=True)

            @pl.loop(0, fire)
            def _(b):
                pltpu.make_async_copy(ones_v, acc_sh.at[idx_v.at[j]],
                                      sem).wait()

        plsc.subcore_barrier()
        pltpu.sync_copy(
            acc_sh.at[pl.ds(s * rows, rows)],
            out_hbm.at[c, pl.ds(s * rows, rows)],
        )

    return deg_kernel(dst2d, ones, zeros)


def _sc_scatter_rows(src2d, dst2d, s_per_w, hs):
    n, d = hs.shape
    npad = _pad_rows(n)
    rows = npad // NS
    zrows = 128
    zeros = jnp.zeros((zrows, d), jnp.float32)
    nseg = 5
    h_seg = s_per_w // nseg
    assert h_seg % 8 == 0 and h_seg * nseg == s_per_w

    @functools.partial(
        pl.kernel,
        out_type=jax.ShapeDtypeStruct((NC, npad, d), jnp.float32),
        mesh=_sc_mesh(),
        scratch_types=[
            pltpu.VMEM_SHARED((npad, d), jnp.float32),
            [pltpu.VMEM((128, d), jnp.float32)] * 2,
            pltpu.VMEM((h_seg, 128), jnp.int32),
            pltpu.VMEM((h_seg, 128), jnp.int32),
            [pltpu.SemaphoreType.DMA] * 2,
            [pltpu.SemaphoreType.DMA] * 2,
        ],
    )
    def scat_kernel(src_hbm, dst_hbm, hs_hbm, zeros_hbm, out_hbm, acc_sh,
                    msgs, src_v, dst_v, gsems, ssems):
        c = lax.axis_index("c")
        s = lax.axis_index("s")
        wid = c * NS + s

        @pl.loop(0, rows // zrows)
        def _(j):
            pltpu.sync_copy(zeros_hbm,
                            acc_sh.at[pl.ds(s * rows + j * zrows, zrows)])

        plsc.subcore_barrier()

        @pl.loop(0, nseg)
        def _(q):
            base = wid * s_per_w + q * h_seg
            pltpu.sync_copy(src_hbm.at[pl.ds(base, h_seg)], src_v)
            pltpu.sync_copy(dst_hbm.at[pl.ds(base, h_seg)], dst_v)
            pltpu.async_copy(hs_hbm.at[src_v.at[0]], msgs[0], gsems[0])
            pltpu.async_copy(hs_hbm.at[src_v.at[1]], msgs[1], gsems[1])

            @pl.loop(0, h_seg, step=2)
            def _(j):
                for b in range(2):
                    cur = j + b
                    pltpu.make_async_copy(hs_hbm.at[src_v.at[cur]], msgs[b],
                                          gsems[b]).wait()
                    pltpu.async_copy(msgs[b], acc_sh.at[dst_v.at[cur]],
                                     ssems[b], add=True)

                    @pl.when(cur + 2 < h_seg)
                    def _():
                        pltpu.make_async_copy(msgs[b], acc_sh.at[dst_v.at[cur]],
                                              ssems[b]).wait()
                        pltpu.async_copy(hs_hbm.at[src_v.at[cur + 2]], msgs[b],
                                         gsems[b])

            for b in range(2):
                pltpu.make_async_copy(msgs[b], acc_sh.at[dst_v.at[0]],
                                      ssems[b]).wait()

        plsc.subcore_barrier()
        pltpu.sync_copy(
            acc_sh.at[pl.ds(s * rows, rows)],
            out_hbm.at[c, pl.ds(s * rows, rows)],
        )

    return scat_kernel(src2d, dst2d, hs, zeros)


_BLK = 1000


def _row_spec(d):
    return pl.BlockSpec((_BLK, d), lambda i: (i, 0))


def _full_spec(shape):
    nd = len(shape)
    return pl.BlockSpec(shape, lambda i, _n=nd: (0,) * _n)


def _tc1_body(x_ref, degp_ref, w0_ref, l0w_ref, l0b_ref, hs0_ref, res0_ref,
              dinv_ref):
    deg = degp_ref[0][:, 0:1] + degp_ref[1][:, 0:1] + 1.0
    dinv = lax.rsqrt(deg)
    x = x_ref[...]
    h0 = jnp.dot(x, w0_ref[...], preferred_element_type=jnp.float32)
    dinv_b = jnp.broadcast_to(dinv, h0.shape)
    hs0_ref[...] = h0 * dinv_b
    res0_ref[...] = (
        jnp.dot(x, l0w_ref[...], preferred_element_type=jnp.float32)
        + l0b_ref[...]
    )
    dinv_ref[...] = dinv_b


def _ln_relu(t, g, b):
    mu = jnp.mean(t, axis=-1, keepdims=True)
    var = jnp.mean((t - mu) ** 2, axis=-1, keepdims=True)
    return jnp.maximum((t - mu) * lax.rsqrt(var + LN_EPS) * g + b, 0.0)


def _tc2_body(accp_ref, hs0_ref, res0_ref, dinv_ref, b0_ref, g0_ref, be0_ref,
              w1_ref, l1w_ref, l1b_ref, hs1_ref, res1_ref):
    dinv_b = dinv_ref[...]
    gcn0 = (accp_ref[0] + accp_ref[1] + hs0_ref[...]) * dinv_b + b0_ref[...]
    h1 = _ln_relu(gcn0 + res0_ref[...], g0_ref[...], be0_ref[...])
    hs1_ref[...] = (
        jnp.dot(h1, w1_ref[...], preferred_element_type=jnp.float32) * dinv_b
    )
    res1_ref[...] = (
        jnp.dot(h1, l1w_ref[...], preferred_element_type=jnp.float32)
        + l1b_ref[...]
    )


def _tc3_body(accp_ref, hs1_ref, res1_ref, dinv_ref, b1_ref, g1_ref, be1_ref,
              out_ref):
    gcn1 = ((accp_ref[0] + accp_ref[1] + hs1_ref[...]) * dinv_ref[...]
            + b1_ref[...])
    out_ref[...] = _ln_relu(gcn1 + res1_ref[...], g1_ref[...], be1_ref[...])


def kernel(x, edge_index, W0, b0, L0W, L0b, g0, be0, W1, b1, L1W, L1b, g1,
           be1):
    n, d = x.shape
    grid = (n // _BLK,)
    row = _row_spec(d)
    mat = _full_spec((d, d))
    vec = _full_spec((1, d))
    f32 = jnp.float32
    rows_out = jax.ShapeDtypeStruct((n, d), f32)

    src = edge_index[0]
    dst = edge_index[1]
    src2d, dst2d, s_per_w = _chunk_indices(src, dst, n, _pad_rows(n))
    degp = _sc_degree(dst2d, s_per_w, n)

    hs0, res0, dinv_b = pl.pallas_call(
        _tc1_body,
        grid=grid,
        in_specs=[
            row,
            pl.BlockSpec((NC, _BLK, d), lambda i: (0, i, 0)),
            mat, mat, vec,
        ],
        out_specs=[row, row, row],
        out_shape=[rows_out, rows_out, rows_out],
    )(x, degp, W0, L0W, L0b.reshape(1, d))

    accp0 = _sc_scatter_rows(src2d, dst2d, s_per_w, hs0)

    hs1, res1 = pl.pallas_call(
        _tc2_body,
        grid=grid,
        in_specs=[
            pl.BlockSpec((NC, _BLK, d), lambda i: (0, i, 0)),
            row, row, row, vec, vec, vec, mat, mat, vec,
        ],
        out_specs=[row, row],
        out_shape=[rows_out, rows_out],
    )(accp0, hs0, res0, dinv_b, b0.reshape(1, d), g0.reshape(1, d),
      be0.reshape(1, d), W1, L1W, L1b.reshape(1, d))

    accp1 = _sc_scatter_rows(src2d, dst2d, s_per_w, hs1)

    out = pl.pallas_call(
        _tc3_body,
        grid=grid,
        in_specs=[
            pl.BlockSpec((NC, _BLK, d), lambda i: (0, i, 0)),
            row, row, row, vec, vec, vec,
        ],
        out_specs=row,
        out_shape=rows_out,
    )(accp1, hs1, res1, dinv_b, b1.reshape(1, d), g1.reshape(1, d),
      be1.reshape(1, d))

    return out
